# baseline (speedup 1.0000x reference)
import functools

import jax
import jax.numpy as jnp
from jax import lax
from jax.experimental import pallas as pl
from jax.experimental.pallas import tpu as pltpu
from jax.experimental.pallas import tpu_sc as plsc

B, C, K = 4096, 26, 1000
NC, NS = 2, 16
NW = NC * NS
RPW = B // NW
L = 16
CT, KT = 4, 8
NT = CT * KT

_mesh = plsc.VectorSubcoreMesh(core_axis_name="c", subcore_axis_name="s")


@functools.partial(
    pl.kernel,
    mesh=_mesh,
    out_type=jax.ShapeDtypeStruct((B, C, K), jnp.int32),
    compiler_params=pltpu.CompilerParams(
        needs_layout_passes=False, disable_bounds_checks=True),
    scratch_types=[
        pltpu.VMEM((RPW * C,), jnp.int32),
        pltpu.VMEM((NT, 8, 128), jnp.int32),
        pltpu.VMEM((NT, 8, 128), jnp.int32),
        pltpu.SemaphoreType.DMA,
        pltpu.SemaphoreType.DMA,
    ],
)
def _onehot_sc(x_hbm, out_hbm, xl, buf_a, buf_b, sem_a, sem_b):
    wid = lax.axis_index("s") * NC + lax.axis_index("c")
    base = wid * RPW

    pltpu.sync_copy(x_hbm.at[pl.ds(base * C, RPW * C)], xl)

    zeros = jnp.zeros((L,), jnp.int32)
    ones = jnp.ones((L,), jnp.int32)
    iota = lax.iota(jnp.int32, L)
    c0 = iota
    c1 = iota + (C - L)

    def zfill(t, _):
        def zfill_sub(ci, _):
            def zfill_chunk(j, _):
                o = pl.multiple_of(j * L, L)
                buf_a[t, ci, pl.ds(o, L)] = zeros
                buf_b[t, ci, pl.ds(o, L)] = zeros
                return 0
            return lax.fori_loop(0, 128 // L, zfill_chunk, 0)
        return lax.fori_loop(0, 8, zfill_sub, 0)

    lax.fori_loop(0, NT, zfill, 0)

    def row_vals(r):
        return (plsc.load_gather(xl, [r * C + iota]),
                plsc.load_gather(xl, [r * C + (C - L) + iota]))

    def scatter(buf, vals, what):
        v0, v1 = vals
        plsc.store_scatter(
            buf, [(c0 >> 3) * KT + (v0 >> 7), c0 & 7, v0 & 127], what)
        plsc.store_scatter(
            buf, [(c1 >> 3) * KT + (v1 >> 7), c1 & 7, v1 & 127], what)

    def fire(buf, vals, row, sem):
        scatter(buf, vals, ones)

        def issue(t, _):
            cs = pl.multiple_of((t // KT) * 8, 8)
            ks = pl.multiple_of((t % KT) * 128, 128)
            pltpu.async_copy(
                buf.at[t],
                out_hbm.at[base + row, pl.ds(cs, 8), pl.ds(ks, 128)], sem)
            return 0

        lax.fori_loop(0, NT, issue, 0)

    def drain(buf, sem):
        def one(t, _):
            pltpu.make_async_copy(
                buf.at[0], out_hbm.at[base, pl.ds(0, 8), pl.ds(0, 128)],
                sem).wait()
            return 0
        lax.fori_loop(0, NT, one, 0)

    va = row_vals(0)
    fire(buf_a, va, 0, sem_a)
    vb = row_vals(1)
    fire(buf_b, vb, 1, sem_b)

    def step(g, carry):
        va, vb = carry
        drain(buf_a, sem_a)
        scatter(buf_a, va, zeros)
        nva = row_vals(2 * g)
        fire(buf_a, nva, 2 * g, sem_a)
        drain(buf_b, sem_b)
        scatter(buf_b, vb, zeros)
        nvb = row_vals(2 * g + 1)
        fire(buf_b, nvb, 2 * g + 1, sem_b)
        return nva, nvb

    lax.fori_loop(1, RPW // 2, step, (va, vb))

    drain(buf_a, sem_a)
    drain(buf_b, sem_b)


def kernel(x):
    return _onehot_sc(x.reshape(B * C))
